# baseline (speedup 1.0000x reference)
import jax
import jax.numpy as jnp
from jax.experimental import pallas as pl
from jax.experimental.pallas import tpu as pltpu


def _copy_body(u_in, i_in, u_out, i_out, u_sem, i_sem):
    uv_in = u_in.reshape(1000, 1000, 16)
    uv_out = u_out.reshape(1000, 1000, 16)
    iv_in = i_in.reshape(100, 1000, 16)
    iv_out = i_out.reshape(100, 1000, 16)
    cu = pltpu.make_async_copy(uv_in, uv_out, u_sem)
    ci = pltpu.make_async_copy(iv_in, iv_out, i_sem)
    cu.start()
    ci.start()
    cu.wait()
    ci.wait()


def kernel(user_emb, item_emb):
    return pl.pallas_call(
        _copy_body,
        in_specs=[
            pl.BlockSpec(memory_space=pltpu.MemorySpace.HBM),
            pl.BlockSpec(memory_space=pltpu.MemorySpace.HBM),
        ],
        out_specs=[
            pl.BlockSpec(memory_space=pltpu.MemorySpace.HBM),
            pl.BlockSpec(memory_space=pltpu.MemorySpace.HBM),
        ],
        out_shape=[
            jax.ShapeDtypeStruct(user_emb.shape, user_emb.dtype),
            jax.ShapeDtypeStruct(item_emb.shape, item_emb.dtype),
        ],
        scratch_shapes=[pltpu.SemaphoreType.DMA, pltpu.SemaphoreType.DMA],
    )(user_emb, item_emb)


# speedup vs baseline: 17.8848x; 17.8848x over previous
import jax
import jax.numpy as jnp
from jax import lax
from jax.experimental import pallas as pl
from jax.experimental.pallas import tpu as pltpu
from jax.experimental.pallas import tpu_sc as plsc

CHUNK = 1000
NW = 32
U_CHUNKS = 1_000_000 // CHUNK
I_CHUNKS = 100_000 // CHUNK


def _sc_copy_body(u_in, i_in, u_out, i_out, buf):
    wid = lax.axis_index("s") * 2 + lax.axis_index("c")

    def copy_strided(src, dst, n_chunks):
        @pl.loop(0, (n_chunks + NW - 1) // NW)
        def _(k):
            c = wid + k * NW

            @pl.when(c < n_chunks)
            def _():
                base = c * CHUNK
                pltpu.sync_copy(src.at[pl.ds(base, CHUNK)], buf)
                pltpu.sync_copy(buf, dst.at[pl.ds(base, CHUNK)])

    copy_strided(u_in, u_out, U_CHUNKS)
    copy_strided(i_in, i_out, I_CHUNKS)


def kernel(user_emb, item_emb):
    run = pl.kernel(
        _sc_copy_body,
        out_type=[
            jax.ShapeDtypeStruct(user_emb.shape, user_emb.dtype),
            jax.ShapeDtypeStruct(item_emb.shape, item_emb.dtype),
        ],
        mesh=plsc.VectorSubcoreMesh(core_axis_name="c", subcore_axis_name="s"),
        scratch_types=[pltpu.VMEM((CHUNK, 16), jnp.float32)],
    )
    u_o, i_o = run(user_emb, item_emb)
    return u_o, i_o
